# baseline (speedup 1.0000x reference)
import functools

import jax
import jax.numpy as jnp
from jax import lax
from jax.experimental import pallas as pl
from jax.experimental.pallas import tpu as pltpu

_EPS = 1e-5
_NEG = -3.0e38
_DX1 = (-4, -2, -1, 0, 1, 2, 4)
_DX2 = (-2, -1, 0, 1, 2)
_DX3 = (-4, -2, 0, 2, 4)
_DXP = (-1, 0, 1)

_PARAMS_1D = pltpu.CompilerParams(
    dimension_semantics=("parallel",),
    vmem_limit_bytes=64 * 1024 * 1024,
)


def _full_spec(shape):
    nd = len(shape)
    return pl.BlockSpec(shape, lambda *_: (0,) * nd)


def _geometry(H, W):
    HW = H * W
    iota = lax.broadcasted_iota(jnp.int32, (HW, 1), 0)
    if W & (W - 1) == 0:
        c = iota & (W - 1)
        r = iota >> (W.bit_length() - 1)
    else:
        c = iota % W
        r = iota // W
    cnt = ((1 + (r > 0).astype(jnp.int32) + (r < H - 1).astype(jnp.int32))
           * (1 + (c > 0).astype(jnp.int32) + (c < W - 1).astype(jnp.int32)))
    inv_cnt = 1.0 / cnt.astype(jnp.float32)
    return c, inv_cnt


def _col_mask(c, dx, W):
    if dx == 0:
        return None
    return (c >= -dx) if dx < 0 else (c < W - dx)


def _fill_base(dst, interior, border, *, pad_s, HW, W=None):
    dst[:pad_s, :] = jnp.full((pad_s, dst.shape[1]), border, jnp.float32)
    dst[pad_s + HW:, :] = jnp.full((pad_s, dst.shape[1]), border, jnp.float32)
    dst[pad_s:pad_s + HW, :] = interior


def _fill_big(big, base, dxs, cidx, Wd, *, pad_s, HW, W=None):
    nl = big.shape[1]
    big[:pad_s, :] = jnp.zeros((pad_s, nl), jnp.bfloat16)
    big[pad_s + HW:, :] = jnp.zeros((pad_s, nl), jnp.bfloat16)
    sw = nl // len(dxs)
    for s, dx in enumerate(dxs):
        v = base[pad_s + dx:pad_s + dx + HW, :]
        m = _col_mask(cidx, dx, Wd)
        if m is not None:
            v = jnp.where(m, v, 0.0)
        big[pad_s:pad_s + HW, s * sw:(s + 1) * sw] = v.astype(jnp.bfloat16)


def _fill_shifted(dst, base, dx, border, mask, *, pad_s, HW, W=None):
    dst[:pad_s, :] = jnp.full((pad_s, dst.shape[1]), border, jnp.float32)
    dst[pad_s + HW:, :] = jnp.full((pad_s, dst.shape[1]), border, jnp.float32)
    v = base[pad_s + dx:pad_s + dx + HW, :]
    if mask is not None:
        v = jnp.where(mask, v, border)
    dst[pad_s:pad_s + HW, :] = v


def _conv_mxu(big, m_ref, K, dil, lo, hi, sw, *, pad_s, W, HW):
    half = (K // 2) * dil
    acc = None
    for kh in range(K):
        dy = kh * dil - half
        off = pad_s + dy * W
        y = jnp.dot(big[off:off + HW, lo * sw:hi * sw], m_ref[kh],
                    preferred_element_type=jnp.float32)
        acc = y if acc is None else acc + y
    return acc


def _avg_mxu(big, mavg_ref, inv_cnt, *, pad_s, W, HW):
    acc = None
    for dy in (-1, 0, 1):
        off = pad_s + dy * W
        y = jnp.dot(big[off:off + HW, :], mavg_ref[...],
                    preferred_element_type=jnp.float32)
        acc = y if acc is None else acc + y
    return acc * inv_cnt


def _max_pool(copies, *, pad_s, W, HW):
    acc = None
    for dy in (-1, 0, 1):
        off = pad_s + dy * W
        for dx in (-1, 0, 1):
            v = copies[dx][off:off + HW, :]
            acc = v if acc is None else jnp.maximum(acc, v)
    return acc


def _transpose_in(x_ref, p, ident, C, HW):
    x2 = jnp.concatenate([x_ref[2 * p], x_ref[2 * p + 1]], axis=0)
    return lax.dot_general(x2, ident[...], (((0,), (0,)), ((), ())),
                           preferred_element_type=jnp.float32)


def _put_stats(stats_ref, p, j, y):
    stats_ref[0, p, 2 * j:2 * j + 1, :] = jnp.sum(y, axis=0, keepdims=True)
    stats_ref[0, p, 2 * j + 1:2 * j + 2, :] = jnp.sum(y * y, axis=0,
                                                      keepdims=True)


def _p1_kernel(x_ref, m_s3, m_s5, m_d3, m_d5, m_avg, ident,
               o_s3a, o_s5a, stats_ref,
               relu_b, raw_b, max_b, max_p, max_m, big_r, big_a,
               *, B, C, H, W, pad_s):
    HW = H * W
    cidx, inv_cnt = _geometry(H, W)
    geo = dict(pad_s=pad_s, W=W, HW=HW)
    m_cp = {0: max_b, 1: max_p, -1: max_m}

    for p in range(B // 2):
        xt = _transpose_in(x_ref, p, ident, C, HW)
        _fill_base(relu_b, jnp.maximum(xt, 0.0), 0.0, **geo)
        _fill_base(raw_b, xt, 0.0, **geo)
        _fill_base(max_b, xt, _NEG, **geo)
        _fill_big(big_r, relu_b, _DX1, cidx, W, **geo)
        _fill_big(big_a, raw_b, _DXP, cidx, W, **geo)
        for dx in (1, -1):
            _fill_shifted(m_cp[dx], max_b, dx, _NEG, _col_mask(cidx, dx, W),
                          **geo)

        for j, (o_ref, m_ref, K, dil, lo, hi) in enumerate(
                ((o_s3a, m_s3, 3, 1, 2, 5),
                 (o_s5a, m_s5, 5, 1, 1, 6),
                 (None, m_d3, 3, 2, 0, 7),
                 (None, m_d5, 5, 2, 0, 7))):
            y = _conv_mxu(big_r, m_ref, K, dil, lo, hi, 2 * C, **geo)
            _put_stats(stats_ref, p, j, y)
            if o_ref is not None:
                o_ref[0, p] = y.astype(jnp.bfloat16)

        _put_stats(stats_ref, p, 4, _max_pool(m_cp, **geo))
        _put_stats(stats_ref, p, 5, _avg_mxu(big_a, m_avg, inv_cnt, **geo))


def _pass1(xf, weights, *, B, N, C, H, W, pad_s):
    HW = H * W
    G = N // B
    P = B // 2
    PADT = HW + 2 * pad_s
    img_spec = pl.BlockSpec((B, C, HW), lambda n: (n, 0, 0))
    t_spec = pl.BlockSpec((1, P, HW, 2 * C), lambda n: (n, 0, 0, 0))
    f32s = lambda nl: pltpu.VMEM((PADT, nl), jnp.float32)
    bf16s = lambda nl: pltpu.VMEM((PADT, nl), jnp.bfloat16)
    return pl.pallas_call(
        functools.partial(_p1_kernel, B=B, C=C, H=H, W=W, pad_s=pad_s),
        grid=(G,),
        in_specs=[img_spec] + [_full_spec(w.shape) for w in weights],
        out_specs=[t_spec, t_spec,
                   pl.BlockSpec((1, P, 16, 2 * C), lambda n: (n, 0, 0, 0))],
        out_shape=[jax.ShapeDtypeStruct((G, P, HW, 2 * C), jnp.bfloat16)] * 2
        + [jax.ShapeDtypeStruct((G, P, 16, 2 * C), jnp.float32)],
        scratch_shapes=[f32s(2 * C)] * 5 + [bf16s(len(_DX1) * 2 * C),
                                            bf16s(len(_DXP) * 2 * C)],
        compiler_params=_PARAMS_1D,
    )(xf, *weights)


def _p2_kernel(y3_ref, y5_ref, m2_s3, m2_s5, bn_ref,
               o_s3, o_s5, stats_ref,
               act_b, big,
               *, B, C, H, W, pad_s):
    HW = H * W
    cidx, _ = _geometry(H, W)
    geo = dict(pad_s=pad_s, W=W, HW=HW)

    for p in range(B // 2):
        for j, (y_ref, m_ref, K, o_ref, lo, hi) in enumerate(
                ((y3_ref, m2_s3, 3, o_s3, 1, 4),
                 (y5_ref, m2_s5, 5, o_s5, 0, 5))):
            y = y_ref[0, p].astype(jnp.float32)
            a = jnp.maximum(y * bn_ref[2 * j:2 * j + 1, :]
                            + bn_ref[2 * j + 1:2 * j + 2, :], 0.0)
            _fill_base(act_b, a, 0.0, **geo)
            _fill_big(big, act_b, _DX2, cidx, W, **geo)
            out = _conv_mxu(big, m_ref, K, 1, lo, hi, 2 * C, **geo)
            _put_stats(stats_ref, p, j, out)
            o_ref[0, p] = out.astype(jnp.bfloat16)


def _pass2(y3, y5, weights, *, B, N, C, H, W, pad_s):
    HW = H * W
    G = N // B
    P = B // 2
    PADT = HW + 2 * pad_s
    t_spec = pl.BlockSpec((1, P, HW, 2 * C), lambda n: (n, 0, 0, 0))
    return pl.pallas_call(
        functools.partial(_p2_kernel, B=B, C=C, H=H, W=W, pad_s=pad_s),
        grid=(G,),
        in_specs=[t_spec, t_spec] + [_full_spec(w.shape) for w in weights],
        out_specs=[t_spec, t_spec,
                   pl.BlockSpec((1, P, 8, 2 * C), lambda n: (n, 0, 0, 0))],
        out_shape=[jax.ShapeDtypeStruct((G, P, HW, 2 * C), jnp.bfloat16)] * 2
        + [jax.ShapeDtypeStruct((G, P, 8, 2 * C), jnp.float32)],
        scratch_shapes=[pltpu.VMEM((PADT, 2 * C), jnp.float32),
                        pltpu.VMEM((PADT, len(_DX2) * 2 * C), jnp.bfloat16)],
        compiler_params=_PARAMS_1D,
    )(y3, y5, *weights)


def _p3_kernel(x_ref, s3_ref, s5_ref, m_d3, m_d5, m_avg, sb_ref, ident,
               o_ref,
               relu_b, raw_b, max_b, max_p, max_m, big_r, big_a,
               *, B, C, H, W, pad_s):
    HW = H * W
    cidx, inv_cnt = _geometry(H, W)
    geo = dict(pad_s=pad_s, W=W, HW=HW)
    m_cp = {0: max_b, 1: max_p, -1: max_m}

    for p in range(B // 2):
        xt = _transpose_in(x_ref, p, ident, C, HW)
        _fill_base(relu_b, jnp.maximum(xt, 0.0), 0.0, **geo)
        _fill_base(raw_b, xt, 0.0, **geo)
        _fill_base(max_b, xt, _NEG, **geo)
        _fill_big(big_r, relu_b, _DX3, cidx, W, **geo)
        _fill_big(big_a, raw_b, _DXP, cidx, W, **geo)
        for dx in (1, -1):
            _fill_shifted(m_cp[dx], max_b, dx, _NEG, _col_mask(cidx, dx, W),
                          **geo)

        sb = sb_ref[...]
        acc = xt * sb[4:5, :] + sb[5:6, :]
        acc = acc + s3_ref[0, p].astype(jnp.float32) * sb[0:1, :]
        acc = acc + s5_ref[0, p].astype(jnp.float32) * sb[1:2, :]
        acc = acc + _max_pool(m_cp, **geo) * sb[2:3, :]
        acc = acc + _avg_mxu(big_a, m_avg, inv_cnt, **geo) * sb[3:4, :]
        acc = acc + _conv_mxu(big_r, m_d3, 3, 2, 1, 4, 2 * C, **geo)
        acc = acc + _conv_mxu(big_r, m_d5, 5, 2, 0, 5, 2 * C, **geo)
        out2 = lax.dot_general(ident[...], acc, (((1,), (1,)), ((), ())),
                               preferred_element_type=jnp.float32)
        o_ref[2 * p] = out2[:C]
        o_ref[2 * p + 1] = out2[C:]


def _pass3(xf, s3, s5, weights, *, B, N, C, H, W, pad_s):
    HW = H * W
    G = N // B
    P = B // 2
    PADT = HW + 2 * pad_s
    img_spec = pl.BlockSpec((B, C, HW), lambda n: (n, 0, 0))
    t_spec = pl.BlockSpec((1, P, HW, 2 * C), lambda n: (n, 0, 0, 0))
    f32s = lambda nl: pltpu.VMEM((PADT, nl), jnp.float32)
    bf16s = lambda nl: pltpu.VMEM((PADT, nl), jnp.bfloat16)
    return pl.pallas_call(
        functools.partial(_p3_kernel, B=B, C=C, H=H, W=W, pad_s=pad_s),
        grid=(G,),
        in_specs=[img_spec, t_spec, t_spec]
        + [_full_spec(w.shape) for w in weights],
        out_specs=img_spec,
        out_shape=jax.ShapeDtypeStruct((N, C, HW), jnp.float32),
        scratch_shapes=[f32s(2 * C)] * 5 + [bf16s(len(_DX3) * 2 * C),
                                            bf16s(len(_DXP) * 2 * C)],
        compiler_params=_PARAMS_1D,
    )(xf, s3, s5, *weights)


def kernel(x, sep3_dw1, sep3_pw1, sep3_dw2, sep3_pw2,
           sep5_dw1, sep5_pw1, sep5_dw2, sep5_pw2,
           dil3_dw, dil3_pw, dil5_dw, dil5_pw, alphas):
    N, C, H, W = x.shape
    HW = H * W
    f32 = jnp.float32
    x = x.astype(f32)
    xf = x.reshape(N, C, HW)
    alphas = jnp.asarray(alphas, f32)

    B = 4 if N % 4 == 0 else 2
    pad_s = ((4 * W + 4 + 7) // 8) * 8

    ident = jnp.eye(2 * C, dtype=f32)

    def pw2(a, scale=None):
        m = a[:, :, 0, 0].astype(f32).T
        if scale is not None:
            m = m * scale[None, :]
        z = jnp.zeros((C, C), f32)
        return jnp.concatenate(
            [jnp.concatenate([m, z], axis=1),
             jnp.concatenate([z, m], axis=1)], axis=0)

    def conv_mats(dw, pw, K, dil, dxs, lo, hi, scale=None):
        wd = dw.reshape(C, K * K).astype(f32)
        pmat = pw2(pw, scale)
        slot = {dx: i for i, dx in enumerate(dxs)}
        half = (K // 2) * dil
        sw = 2 * C
        rows = []
        for kh in range(K):
            m = jnp.zeros((len(dxs) * sw, sw), f32)
            for kw in range(K):
                dx = kw * dil - half
                s = slot[dx]
                wcol = jnp.tile(wd[:, kh * K + kw], 2)
                m = m.at[s * sw:(s + 1) * sw, :].set(wcol[:, None] * pmat)
            rows.append(m[lo * sw:hi * sw, :])
        return jnp.stack(rows).astype(jnp.bfloat16)

    m_avg = jnp.tile(jnp.eye(2 * C, dtype=f32), (len(_DXP), 1)) \
        .astype(jnp.bfloat16)

    w1 = (conv_mats(sep3_dw1, sep3_pw1, 3, 1, _DX1, 2, 5),
          conv_mats(sep5_dw1, sep5_pw1, 5, 1, _DX1, 1, 6),
          conv_mats(dil3_dw, dil3_pw, 3, 2, _DX1, 0, 7),
          conv_mats(dil5_dw, dil5_pw, 5, 2, _DX1, 0, 7),
          m_avg, ident)
    y_s3a, y_s5a, stats1 = _pass1(xf, w1, B=B, N=N, C=C, H=H, W=W,
                                  pad_s=pad_s)

    total = jnp.float32(N * HW)
    st1 = jnp.sum(stats1, axis=(0, 1)).reshape(16, 2, C).sum(axis=1)

    def finalize(st, j):
        s, ss = st[2 * j], st[2 * j + 1]
        m = s / total
        v = jnp.maximum(ss / total - m * m, 0.0)
        sc = lax.rsqrt(v + _EPS)
        return sc, -m * sc

    t2 = lambda v: jnp.tile(v, 2)
    sc_s3a, bi_s3a = finalize(st1, 0)
    sc_s5a, bi_s5a = finalize(st1, 1)
    bn_mid = jnp.stack([t2(sc_s3a), t2(bi_s3a), t2(sc_s5a), t2(bi_s5a)])

    w2 = (conv_mats(sep3_dw2, sep3_pw2, 3, 1, _DX2, 1, 4),
          conv_mats(sep5_dw2, sep5_pw2, 5, 1, _DX2, 0, 5), bn_mid)
    y_s3, y_s5, stats2 = _pass2(y_s3a, y_s5a, w2, B=B, N=N, C=C, H=H, W=W,
                                pad_s=pad_s)
    st2 = jnp.sum(stats2, axis=(0, 1)).reshape(8, 2, C).sum(axis=1)

    sc_d3, bi_d3 = finalize(st1, 2)
    sc_d5, bi_d5 = finalize(st1, 3)
    sc_mx, bi_mx = finalize(st1, 4)
    sc_av, bi_av = finalize(st1, 5)
    sc_s3, bi_s3 = finalize(st2, 0)
    sc_s5, bi_s5 = finalize(st2, 1)

    total_bias = (alphas[0] * bi_mx + alphas[1] * bi_av
                  + alphas[3] * bi_s3 + alphas[4] * bi_s5
                  + alphas[5] * bi_d3 + alphas[6] * bi_d5)
    sb = jnp.stack([t2(alphas[3] * sc_s3), t2(alphas[4] * sc_s5),
                    t2(alphas[0] * sc_mx), t2(alphas[1] * sc_av),
                    jnp.full((2 * C,), alphas[2], f32), t2(total_bias),
                    jnp.zeros((2 * C,), f32), jnp.zeros((2 * C,), f32)])

    w3 = (conv_mats(dil3_dw, dil3_pw, 3, 2, _DX3, 1, 4,
                    scale=alphas[5] * sc_d3),
          conv_mats(dil5_dw, dil5_pw, 5, 2, _DX3, 0, 5,
                    scale=alphas[6] * sc_d5),
          m_avg, sb, ident)
    out = _pass3(xf, y_s3, y_s5, w3, B=B, N=N, C=C, H=H, W=W, pad_s=pad_s)
    return out.reshape(N, C, H, W)


# speedup vs baseline: 2.0442x; 2.0442x over previous
import functools

import jax
import jax.numpy as jnp
from jax import lax
from jax.experimental import pallas as pl
from jax.experimental.pallas import tpu as pltpu

_EPS = 1e-5
_NEG = -3.0e38

_PARAMS_1D = pltpu.CompilerParams(
    dimension_semantics=("parallel",),
    vmem_limit_bytes=64 * 1024 * 1024,
)


def _full_spec(shape):
    nd = len(shape)
    return pl.BlockSpec(shape, lambda *_: (0,) * nd)


def _geometry(H, W):
    HW = H * W
    iota = lax.broadcasted_iota(jnp.int32, (HW, 1), 0)
    if W & (W - 1) == 0:
        c = iota & (W - 1)
        r = iota >> (W.bit_length() - 1)
    else:
        c = iota % W
        r = iota // W
    cnt = ((1 + (r > 0).astype(jnp.int32) + (r < H - 1).astype(jnp.int32))
           * (1 + (c > 0).astype(jnp.int32) + (c < W - 1).astype(jnp.int32)))
    inv_cnt = 1.0 / cnt.astype(jnp.float32)
    return c, inv_cnt


def _col_mask(c, dx, W):
    if dx == 0:
        return None
    return (c >= -dx) if dx < 0 else (c < W - dx)


def _fill_shifted(dst, base, dx, border, mask, *, pad_s, HW):
    dst[:pad_s, :] = jnp.full((pad_s, dst.shape[1]), border, jnp.float32)
    dst[pad_s + HW:, :] = jnp.full((pad_s, dst.shape[1]), border, jnp.float32)
    v = base[pad_s + dx:pad_s + dx + HW, :]
    if mask is not None:
        v = jnp.where(mask, v, border)
    dst[pad_s:pad_s + HW, :] = v


def _fill_base(dst, interior, border, *, pad_s, HW):
    dst[:pad_s, :] = jnp.full((pad_s, dst.shape[1]), border, jnp.float32)
    dst[pad_s + HW:, :] = jnp.full((pad_s, dst.shape[1]), border, jnp.float32)
    dst[pad_s:pad_s + HW, :] = interior


def _conv_t(copies, wdw_ref, K, dil, *, pad_s, W, HW):
    half = (K // 2) * dil
    acc = None
    for kh in range(K):
        dy = kh * dil - half
        off = pad_s + dy * W
        for kw in range(K):
            dx = kw * dil - half
            v = copies[dx][off:off + HW, :]
            t = v * wdw_ref[kh * K + kw:kh * K + kw + 1, :]
            acc = t if acc is None else acc + t
    return acc


def _pool_t(copies, op, *, pad_s, W, HW):
    acc = None
    for dy in (-1, 0, 1):
        off = pad_s + dy * W
        for dx in (-1, 0, 1):
            v = copies[dx][off:off + HW, :]
            acc = v if acc is None else op(acc, v)
    return acc


def _transpose_in(x_ref, p, ident, C, HW):
    x2 = jnp.concatenate([x_ref[2 * p], x_ref[2 * p + 1]], axis=0)
    return lax.dot_general(x2, ident[...], (((0,), (0,)), ((), ())),
                           preferred_element_type=jnp.float32)


def _p1_kernel(x_ref,
               wdw_s3, wpw_s3, wdw_s5, wpw_s5,
               wdw_d3, wpw_d3, wdw_d5, wpw_d5, ident,
               o_s3a, o_s5a, o_d3, o_d5, o_mx, o_av, stats_ref,
               *scr,
               B, C, H, W, pad_s):
    HW = H * W
    cidx, inv_cnt = _geometry(H, W)
    geo = dict(pad_s=pad_s, W=W, HW=HW)
    r_cp = {0: scr[0], 1: scr[1], -1: scr[2], 2: scr[3], -2: scr[4],
            4: scr[5], -4: scr[6]}
    a_cp = {0: scr[7], 1: scr[8], -1: scr[9]}
    m_cp = {0: scr[10], 1: scr[11], -1: scr[12]}

    for p in range(B // 2):
        xt = _transpose_in(x_ref, p, ident, C, HW)
        _fill_base(scr[0], jnp.maximum(xt, 0.0), 0.0, pad_s=pad_s, HW=HW)
        _fill_base(scr[7], xt, 0.0, pad_s=pad_s, HW=HW)
        _fill_base(scr[10], xt, _NEG, pad_s=pad_s, HW=HW)
        for dx in (1, -1, 2, -2, 4, -4):
            _fill_shifted(r_cp[dx], scr[0], dx, 0.0, _col_mask(cidx, dx, W),
                          pad_s=pad_s, HW=HW)
        for dx in (1, -1):
            _fill_shifted(a_cp[dx], scr[7], dx, 0.0, _col_mask(cidx, dx, W),
                          pad_s=pad_s, HW=HW)
            _fill_shifted(m_cp[dx], scr[10], dx, _NEG,
                          _col_mask(cidx, dx, W), pad_s=pad_s, HW=HW)

        for j, (o_ref, wdw, wpw, K, dil) in enumerate(
                ((o_s3a, wdw_s3, wpw_s3, 3, 1),
                 (o_s5a, wdw_s5, wpw_s5, 5, 1),
                 (o_d3, wdw_d3, wpw_d3, 3, 2),
                 (o_d5, wdw_d5, wpw_d5, 5, 2))):
            dw = _conv_t(r_cp, wdw, K, dil, **geo)
            y = jnp.dot(dw, wpw[...], preferred_element_type=jnp.float32)
            stats_ref[0, p, 2 * j:2 * j + 1, :] = jnp.sum(y, axis=0,
                                                          keepdims=True)
            stats_ref[0, p, 2 * j + 1:2 * j + 2, :] = jnp.sum(
                y * y, axis=0, keepdims=True)
            o_ref[0, p] = y.astype(jnp.bfloat16)

        mx = _pool_t(m_cp, jnp.maximum, **geo)
        av = _pool_t(a_cp, jnp.add, **geo) * inv_cnt
        for j, o_ref, y in ((4, o_mx, mx), (5, o_av, av)):
            stats_ref[0, p, 2 * j:2 * j + 1, :] = jnp.sum(y, axis=0,
                                                          keepdims=True)
            stats_ref[0, p, 2 * j + 1:2 * j + 2, :] = jnp.sum(
                y * y, axis=0, keepdims=True)
            o_ref[0, p] = y.astype(jnp.bfloat16)


def _pass1(xf, weights, *, B, N, C, H, W, pad_s):
    HW = H * W
    G = N // B
    P = B // 2
    PADT = HW + 2 * pad_s
    img_spec = pl.BlockSpec((B, C, HW), lambda n: (n, 0, 0))
    t_spec = pl.BlockSpec((1, P, HW, 2 * C), lambda n: (n, 0, 0, 0))
    return pl.pallas_call(
        functools.partial(_p1_kernel, B=B, C=C, H=H, W=W, pad_s=pad_s),
        grid=(G,),
        in_specs=[img_spec] + [_full_spec(w.shape) for w in weights],
        out_specs=[t_spec] * 6
        + [pl.BlockSpec((1, P, 16, 2 * C), lambda n: (n, 0, 0, 0))],
        out_shape=[jax.ShapeDtypeStruct((G, P, HW, 2 * C), jnp.bfloat16)] * 6
        + [jax.ShapeDtypeStruct((G, P, 16, 2 * C), jnp.float32)],
        scratch_shapes=[pltpu.VMEM((PADT, 2 * C), jnp.float32)] * 13,
        compiler_params=_PARAMS_1D,
    )(xf, *weights)


def _p2_kernel(y3_ref, y5_ref,
               wdw2_s3, wpw2_s3, wdw2_s5, wpw2_s5, bn_ref,
               o_s3, o_s5, stats_ref,
               *scr,
               B, C, H, W, pad_s):
    HW = H * W
    cidx, _ = _geometry(H, W)
    geo = dict(pad_s=pad_s, W=W, HW=HW)

    for p in range(B // 2):
        for j, (y_ref, wdw, wpw, K, o_ref) in enumerate(
                ((y3_ref, wdw2_s3, wpw2_s3, 3, o_s3),
                 (y5_ref, wdw2_s5, wpw2_s5, 5, o_s5))):
            y = y_ref[0, p].astype(jnp.float32)
            a = jnp.maximum(y * bn_ref[2 * j:2 * j + 1, :]
                            + bn_ref[2 * j + 1:2 * j + 2, :], 0.0)
            cp = {0: scr[0], 1: scr[1], -1: scr[2], 2: scr[3], -2: scr[4]}
            _fill_base(scr[0], a, 0.0, pad_s=pad_s, HW=HW)
            dxs = (1, -1) if K == 3 else (1, -1, 2, -2)
            for dx in dxs:
                _fill_shifted(cp[dx], scr[0], dx, 0.0,
                              _col_mask(cidx, dx, W), pad_s=pad_s, HW=HW)
            dw = _conv_t(cp, wdw, K, 1, **geo)
            out = jnp.dot(dw, wpw[...], preferred_element_type=jnp.float32)
            stats_ref[0, p, 2 * j:2 * j + 1, :] = jnp.sum(out, axis=0,
                                                          keepdims=True)
            stats_ref[0, p, 2 * j + 1:2 * j + 2, :] = jnp.sum(
                out * out, axis=0, keepdims=True)
            o_ref[0, p] = out.astype(jnp.bfloat16)


def _pass2(y3, y5, weights, *, B, N, C, H, W, pad_s):
    HW = H * W
    G = N // B
    P = B // 2
    PADT = HW + 2 * pad_s
    t_spec = pl.BlockSpec((1, P, HW, 2 * C), lambda n: (n, 0, 0, 0))
    return pl.pallas_call(
        functools.partial(_p2_kernel, B=B, C=C, H=H, W=W, pad_s=pad_s),
        grid=(G,),
        in_specs=[t_spec, t_spec] + [_full_spec(w.shape) for w in weights],
        out_specs=[t_spec, t_spec,
                   pl.BlockSpec((1, P, 8, 2 * C), lambda n: (n, 0, 0, 0))],
        out_shape=[jax.ShapeDtypeStruct((G, P, HW, 2 * C), jnp.bfloat16)] * 2
        + [jax.ShapeDtypeStruct((G, P, 8, 2 * C), jnp.float32)],
        scratch_shapes=[pltpu.VMEM((PADT, 2 * C), jnp.float32)] * 5,
        compiler_params=_PARAMS_1D,
    )(y3, y5, *weights)


def _p3_kernel(x_ref, s3_ref, s5_ref, d3_ref, d5_ref, mx_ref, av_ref,
               sb_ref, ident, o_ref,
               *, B, C, H, W, pad_s):
    HW = H * W
    sb = sb_ref[...]
    for p in range(B // 2):
        xt = _transpose_in(x_ref, p, ident, C, HW)
        acc = xt * sb[4:5, :] + sb[5:6, :]
        for row, ref in ((0, s3_ref), (1, s5_ref), (6, d3_ref), (7, d5_ref),
                         (2, mx_ref), (3, av_ref)):
            acc = acc + ref[0, p].astype(jnp.float32) * sb[row:row + 1, :]
        out2 = lax.dot_general(ident[...], acc, (((1,), (1,)), ((), ())),
                               preferred_element_type=jnp.float32)
        o_ref[2 * p] = out2[:C]
        o_ref[2 * p + 1] = out2[C:]


def _pass3(xf, branches, weights, *, B, N, C, H, W, pad_s):
    HW = H * W
    G = N // B
    P = B // 2
    img_spec = pl.BlockSpec((B, C, HW), lambda n: (n, 0, 0))
    t_spec = pl.BlockSpec((1, P, HW, 2 * C), lambda n: (n, 0, 0, 0))
    return pl.pallas_call(
        functools.partial(_p3_kernel, B=B, C=C, H=H, W=W, pad_s=pad_s),
        grid=(G,),
        in_specs=[img_spec] + [t_spec] * 6
        + [_full_spec(w.shape) for w in weights],
        out_specs=img_spec,
        out_shape=jax.ShapeDtypeStruct((N, C, HW), jnp.float32),
        compiler_params=_PARAMS_1D,
    )(xf, *branches, *weights)


def kernel(x, sep3_dw1, sep3_pw1, sep3_dw2, sep3_pw2,
           sep5_dw1, sep5_pw1, sep5_dw2, sep5_pw2,
           dil3_dw, dil3_pw, dil5_dw, dil5_pw, alphas):
    N, C, H, W = x.shape
    HW = H * W
    f32 = jnp.float32
    x = x.astype(f32)
    xf = x.reshape(N, C, HW)
    alphas = jnp.asarray(alphas, f32)

    B = 4 if N % 4 == 0 else 2
    pad_s = ((4 * W + 4 + 7) // 8) * 8

    ident = jnp.eye(2 * C, dtype=f32)

    def dwT(a):
        return jnp.tile(a.reshape(C, -1).astype(f32).T, (1, 2))

    def pwT(a, scale=None):
        m = a[:, :, 0, 0].astype(f32).T
        if scale is not None:
            m = m * scale[None, :]
        z = jnp.zeros((C, C), f32)
        return jnp.concatenate(
            [jnp.concatenate([m, z], axis=1),
             jnp.concatenate([z, m], axis=1)], axis=0)

    w1 = (dwT(sep3_dw1), pwT(sep3_pw1), dwT(sep5_dw1), pwT(sep5_pw1),
          dwT(dil3_dw), pwT(dil3_pw), dwT(dil5_dw), pwT(dil5_pw), ident)
    (y_s3a, y_s5a, y_d3, y_d5, y_mx, y_av, stats1) = _pass1(
        xf, w1, B=B, N=N, C=C, H=H, W=W, pad_s=pad_s)

    total = jnp.float32(N * HW)
    st1 = jnp.sum(stats1, axis=(0, 1)).reshape(16, 2, C).sum(axis=1)

    def finalize(st, j):
        s, ss = st[2 * j], st[2 * j + 1]
        m = s / total
        v = jnp.maximum(ss / total - m * m, 0.0)
        sc = lax.rsqrt(v + _EPS)
        return sc, -m * sc

    t2 = lambda v: jnp.tile(v, 2)
    sc_s3a, bi_s3a = finalize(st1, 0)
    sc_s5a, bi_s5a = finalize(st1, 1)
    bn_mid = jnp.stack([t2(sc_s3a), t2(bi_s3a), t2(sc_s5a), t2(bi_s5a)])

    w2 = (dwT(sep3_dw2), pwT(sep3_pw2), dwT(sep5_dw2), pwT(sep5_pw2), bn_mid)
    y_s3, y_s5, stats2 = _pass2(y_s3a, y_s5a, w2, B=B, N=N, C=C, H=H, W=W,
                                pad_s=pad_s)
    st2 = jnp.sum(stats2, axis=(0, 1)).reshape(8, 2, C).sum(axis=1)

    sc_d3, bi_d3 = finalize(st1, 2)
    sc_d5, bi_d5 = finalize(st1, 3)
    sc_mx, bi_mx = finalize(st1, 4)
    sc_av, bi_av = finalize(st1, 5)
    sc_s3, bi_s3 = finalize(st2, 0)
    sc_s5, bi_s5 = finalize(st2, 1)

    total_bias = (alphas[0] * bi_mx + alphas[1] * bi_av
                  + alphas[3] * bi_s3 + alphas[4] * bi_s5
                  + alphas[5] * bi_d3 + alphas[6] * bi_d5)
    sb = jnp.stack([t2(alphas[3] * sc_s3), t2(alphas[4] * sc_s5),
                    t2(alphas[0] * sc_mx), t2(alphas[1] * sc_av),
                    jnp.full((2 * C,), alphas[2], f32), t2(total_bias),
                    t2(alphas[5] * sc_d3), t2(alphas[6] * sc_d5)])

    out = _pass3(xf, (y_s3, y_s5, y_d3, y_d5, y_mx, y_av), (sb, ident),
                 B=B, N=N, C=C, H=H, W=W, pad_s=pad_s)
    return out.reshape(N, C, H, W)


# speedup vs baseline: 2.0636x; 1.0095x over previous
import functools

import jax
import jax.numpy as jnp
from jax import lax
from jax.experimental import pallas as pl
from jax.experimental.pallas import tpu as pltpu

_EPS = 1e-5
_NEG = -3.0e38

_PARAMS_1D = pltpu.CompilerParams(
    dimension_semantics=("parallel",),
    vmem_limit_bytes=64 * 1024 * 1024,
)


def _full_spec(shape):
    nd = len(shape)
    return pl.BlockSpec(shape, lambda *_: (0,) * nd)


def _geometry(H, W):
    HW = H * W
    iota = lax.broadcasted_iota(jnp.int32, (HW, 1), 0)
    if W & (W - 1) == 0:
        c = iota & (W - 1)
        r = iota >> (W.bit_length() - 1)
    else:
        c = iota % W
        r = iota // W
    cnt = ((1 + (r > 0).astype(jnp.int32) + (r < H - 1).astype(jnp.int32))
           * (1 + (c > 0).astype(jnp.int32) + (c < W - 1).astype(jnp.int32)))
    inv_cnt = 1.0 / cnt.astype(jnp.float32)
    return c, inv_cnt


def _col_mask(c, dx, W):
    if dx == 0:
        return None
    return (c >= -dx) if dx < 0 else (c < W - dx)


def _fill_shifted(dst, base, dx, border, mask, *, pad_s, HW):
    dst[:pad_s, :] = jnp.full((pad_s, dst.shape[1]), border, jnp.float32)
    dst[pad_s + HW:, :] = jnp.full((pad_s, dst.shape[1]), border, jnp.float32)
    v = base[pad_s + dx:pad_s + dx + HW, :]
    if mask is not None:
        v = jnp.where(mask, v, border)
    dst[pad_s:pad_s + HW, :] = v


def _fill_base(dst, interior, border, *, pad_s, HW):
    dst[:pad_s, :] = jnp.full((pad_s, dst.shape[1]), border, jnp.float32)
    dst[pad_s + HW:, :] = jnp.full((pad_s, dst.shape[1]), border, jnp.float32)
    dst[pad_s:pad_s + HW, :] = interior


def _conv_t(copies, wdw_ref, K, dil, *, pad_s, W, HW):
    half = (K // 2) * dil
    acc = None
    for kh in range(K):
        dy = kh * dil - half
        off = pad_s + dy * W
        for kw in range(K):
            dx = kw * dil - half
            v = copies[dx][off:off + HW, :]
            t = v * wdw_ref[kh * K + kw:kh * K + kw + 1, :]
            acc = t if acc is None else acc + t
    return acc


def _pool_t(copies, op, *, pad_s, W, HW):
    acc = None
    for dy in (-1, 0, 1):
        off = pad_s + dy * W
        for dx in (-1, 0, 1):
            v = copies[dx][off:off + HW, :]
            acc = v if acc is None else op(acc, v)
    return acc


def _transpose_in(x_ref, p, ident, C, HW):
    x2 = x_ref[2 * p:2 * p + 2].reshape(2 * C, HW)
    return lax.dot_general(x2, ident[...], (((0,), (0,)), ((), ())),
                           preferred_element_type=jnp.float32)


def _sums(y, ones_row):
    s = lax.dot_general(ones_row, y, (((1,), (0,)), ((), ())),
                        preferred_element_type=jnp.float32)
    ss = lax.dot_general(ones_row, y * y, (((1,), (0,)), ((), ())),
                         preferred_element_type=jnp.float32)
    return s, ss


def _p1_kernel(x_ref,
               wdw_s3, wpw_s3, wdw_s5, wpw_s5,
               wdw_d3, wpw_d3, wdw_d5, wpw_d5, ident,
               o_s3a, o_s5a, o_d3, o_d5, o_mx, o_av, stats_ref,
               *scr,
               B, C, H, W, pad_s):
    HW = H * W
    cidx, inv_cnt = _geometry(H, W)
    ones_row = jnp.ones((1, HW), jnp.float32)
    geo = dict(pad_s=pad_s, W=W, HW=HW)
    r_cp = {0: scr[0], 1: scr[1], -1: scr[2], 2: scr[3], -2: scr[4],
            4: scr[5], -4: scr[6]}
    a_cp = {0: scr[7], 1: scr[8], -1: scr[9]}
    m_cp = {0: scr[10], 1: scr[11], -1: scr[12]}

    for p in range(B // 2):
        xt = _transpose_in(x_ref, p, ident, C, HW)
        _fill_base(scr[0], jnp.maximum(xt, 0.0), 0.0, pad_s=pad_s, HW=HW)
        _fill_base(scr[7], xt, 0.0, pad_s=pad_s, HW=HW)
        _fill_base(scr[10], xt, _NEG, pad_s=pad_s, HW=HW)
        for dx in (1, -1, 2, -2, 4, -4):
            _fill_shifted(r_cp[dx], scr[0], dx, 0.0, _col_mask(cidx, dx, W),
                          pad_s=pad_s, HW=HW)
        for dx in (1, -1):
            _fill_shifted(a_cp[dx], scr[7], dx, 0.0, _col_mask(cidx, dx, W),
                          pad_s=pad_s, HW=HW)
            _fill_shifted(m_cp[dx], scr[10], dx, _NEG,
                          _col_mask(cidx, dx, W), pad_s=pad_s, HW=HW)

        for j, (o_ref, wdw, wpw, K, dil) in enumerate(
                ((o_s3a, wdw_s3, wpw_s3, 3, 1),
                 (o_s5a, wdw_s5, wpw_s5, 5, 1),
                 (o_d3, wdw_d3, wpw_d3, 3, 2),
                 (o_d5, wdw_d5, wpw_d5, 5, 2))):
            dw = _conv_t(r_cp, wdw, K, dil, **geo)
            y = jnp.dot(dw, wpw[...], preferred_element_type=jnp.float32)
            s, ss = _sums(y, ones_row)
            stats_ref[0, p, 2 * j:2 * j + 1, :] = s
            stats_ref[0, p, 2 * j + 1:2 * j + 2, :] = ss
            o_ref[0, p] = y.astype(jnp.bfloat16)

        mx = _pool_t(m_cp, jnp.maximum, **geo)
        av = _pool_t(a_cp, jnp.add, **geo) * inv_cnt
        for j, o_ref, y in ((4, o_mx, mx), (5, o_av, av)):
            s, ss = _sums(y, ones_row)
            stats_ref[0, p, 2 * j:2 * j + 1, :] = s
            stats_ref[0, p, 2 * j + 1:2 * j + 2, :] = ss
            o_ref[0, p] = y.astype(jnp.bfloat16)


def _pass1(xf, weights, *, B, N, C, H, W, pad_s):
    HW = H * W
    G = N // B
    P = B // 2
    PADT = HW + 2 * pad_s
    img_spec = pl.BlockSpec((B, C, HW), lambda n: (n, 0, 0))
    t_spec = pl.BlockSpec((1, P, HW, 2 * C), lambda n: (n, 0, 0, 0))
    return pl.pallas_call(
        functools.partial(_p1_kernel, B=B, C=C, H=H, W=W, pad_s=pad_s),
        grid=(G,),
        in_specs=[img_spec] + [_full_spec(w.shape) for w in weights],
        out_specs=[t_spec] * 6
        + [pl.BlockSpec((1, P, 16, 2 * C), lambda n: (n, 0, 0, 0))],
        out_shape=[jax.ShapeDtypeStruct((G, P, HW, 2 * C), jnp.bfloat16)] * 6
        + [jax.ShapeDtypeStruct((G, P, 16, 2 * C), jnp.float32)],
        scratch_shapes=[pltpu.VMEM((PADT, 2 * C), jnp.float32)] * 13,
        compiler_params=_PARAMS_1D,
    )(xf, *weights)


def _p2_kernel(y3_ref, y5_ref,
               wdw2_s3, wpw2_s3, wdw2_s5, wpw2_s5, bn_ref,
               o_s3, o_s5, stats_ref,
               *scr,
               B, C, H, W, pad_s):
    HW = H * W
    cidx, _ = _geometry(H, W)
    ones_row = jnp.ones((1, HW), jnp.float32)
    geo = dict(pad_s=pad_s, W=W, HW=HW)

    for p in range(B // 2):
        for j, (y_ref, wdw, wpw, K, o_ref) in enumerate(
                ((y3_ref, wdw2_s3, wpw2_s3, 3, o_s3),
                 (y5_ref, wdw2_s5, wpw2_s5, 5, o_s5))):
            y = y_ref[0, p].astype(jnp.float32)
            a = jnp.maximum(y * bn_ref[2 * j:2 * j + 1, :]
                            + bn_ref[2 * j + 1:2 * j + 2, :], 0.0)
            cp = {0: scr[0], 1: scr[1], -1: scr[2], 2: scr[3], -2: scr[4]}
            _fill_base(scr[0], a, 0.0, pad_s=pad_s, HW=HW)
            dxs = (1, -1) if K == 3 else (1, -1, 2, -2)
            for dx in dxs:
                _fill_shifted(cp[dx], scr[0], dx, 0.0,
                              _col_mask(cidx, dx, W), pad_s=pad_s, HW=HW)
            dw = _conv_t(cp, wdw, K, 1, **geo)
            out = jnp.dot(dw, wpw[...], preferred_element_type=jnp.float32)
            s, ss = _sums(out, ones_row)
            stats_ref[0, p, 2 * j:2 * j + 1, :] = s
            stats_ref[0, p, 2 * j + 1:2 * j + 2, :] = ss
            o_ref[0, p] = out.astype(jnp.bfloat16)


def _pass2(y3, y5, weights, *, B, N, C, H, W, pad_s):
    HW = H * W
    G = N // B
    P = B // 2
    PADT = HW + 2 * pad_s
    t_spec = pl.BlockSpec((1, P, HW, 2 * C), lambda n: (n, 0, 0, 0))
    return pl.pallas_call(
        functools.partial(_p2_kernel, B=B, C=C, H=H, W=W, pad_s=pad_s),
        grid=(G,),
        in_specs=[t_spec, t_spec] + [_full_spec(w.shape) for w in weights],
        out_specs=[t_spec, t_spec,
                   pl.BlockSpec((1, P, 8, 2 * C), lambda n: (n, 0, 0, 0))],
        out_shape=[jax.ShapeDtypeStruct((G, P, HW, 2 * C), jnp.bfloat16)] * 2
        + [jax.ShapeDtypeStruct((G, P, 8, 2 * C), jnp.float32)],
        scratch_shapes=[pltpu.VMEM((PADT, 2 * C), jnp.float32)] * 5,
        compiler_params=_PARAMS_1D,
    )(y3, y5, *weights)


def _p3_kernel(x_ref, s3_ref, s5_ref, d3_ref, d5_ref, mx_ref, av_ref,
               sb_ref, ident, o_ref,
               *, B, C, H, W, pad_s):
    HW = H * W
    sb = sb_ref[...]
    for p in range(B // 2):
        xt = _transpose_in(x_ref, p, ident, C, HW)
        acc = xt * sb[4:5, :] + sb[5:6, :]
        for row, ref in ((0, s3_ref), (1, s5_ref), (6, d3_ref), (7, d5_ref),
                         (2, mx_ref), (3, av_ref)):
            acc = acc + ref[0, p].astype(jnp.float32) * sb[row:row + 1, :]
        out2 = lax.dot_general(ident[...], acc, (((1,), (1,)), ((), ())),
                               preferred_element_type=jnp.float32)
        o_ref[2 * p] = out2[:C]
        o_ref[2 * p + 1] = out2[C:]


def _pass3(xf, branches, weights, *, B, N, C, H, W, pad_s):
    HW = H * W
    G = N // B
    P = B // 2
    img_spec = pl.BlockSpec((B, C, HW), lambda n: (n, 0, 0))
    t_spec = pl.BlockSpec((1, P, HW, 2 * C), lambda n: (n, 0, 0, 0))
    return pl.pallas_call(
        functools.partial(_p3_kernel, B=B, C=C, H=H, W=W, pad_s=pad_s),
        grid=(G,),
        in_specs=[img_spec] + [t_spec] * 6
        + [_full_spec(w.shape) for w in weights],
        out_specs=img_spec,
        out_shape=jax.ShapeDtypeStruct((N, C, HW), jnp.float32),
        compiler_params=_PARAMS_1D,
    )(xf, *branches, *weights)


def kernel(x, sep3_dw1, sep3_pw1, sep3_dw2, sep3_pw2,
           sep5_dw1, sep5_pw1, sep5_dw2, sep5_pw2,
           dil3_dw, dil3_pw, dil5_dw, dil5_pw, alphas):
    N, C, H, W = x.shape
    HW = H * W
    f32 = jnp.float32
    x = x.astype(f32)
    xf = x.reshape(N, C, HW)
    alphas = jnp.asarray(alphas, f32)

    B = 4 if N % 4 == 0 else 2
    pad_s = ((4 * W + 4 + 7) // 8) * 8

    ident = jnp.eye(2 * C, dtype=f32)

    def dwT(a):
        return jnp.tile(a.reshape(C, -1).astype(f32).T, (1, 2))

    def pwT(a, scale=None):
        m = a[:, :, 0, 0].astype(f32).T
        if scale is not None:
            m = m * scale[None, :]
        z = jnp.zeros((C, C), f32)
        return jnp.concatenate(
            [jnp.concatenate([m, z], axis=1),
             jnp.concatenate([z, m], axis=1)], axis=0)

    w1 = (dwT(sep3_dw1), pwT(sep3_pw1), dwT(sep5_dw1), pwT(sep5_pw1),
          dwT(dil3_dw), pwT(dil3_pw), dwT(dil5_dw), pwT(dil5_pw), ident)
    (y_s3a, y_s5a, y_d3, y_d5, y_mx, y_av, stats1) = _pass1(
        xf, w1, B=B, N=N, C=C, H=H, W=W, pad_s=pad_s)

    total = jnp.float32(N * HW)
    st1 = jnp.sum(stats1, axis=(0, 1)).reshape(16, 2, C).sum(axis=1)

    def finalize(st, j):
        s, ss = st[2 * j], st[2 * j + 1]
        m = s / total
        v = jnp.maximum(ss / total - m * m, 0.0)
        sc = lax.rsqrt(v + _EPS)
        return sc, -m * sc

    t2 = lambda v: jnp.tile(v, 2)
    sc_s3a, bi_s3a = finalize(st1, 0)
    sc_s5a, bi_s5a = finalize(st1, 1)
    bn_mid = jnp.stack([t2(sc_s3a), t2(bi_s3a), t2(sc_s5a), t2(bi_s5a)])

    w2 = (dwT(sep3_dw2), pwT(sep3_pw2), dwT(sep5_dw2), pwT(sep5_pw2), bn_mid)
    y_s3, y_s5, stats2 = _pass2(y_s3a, y_s5a, w2, B=B, N=N, C=C, H=H, W=W,
                                pad_s=pad_s)
    st2 = jnp.sum(stats2, axis=(0, 1)).reshape(8, 2, C).sum(axis=1)

    sc_d3, bi_d3 = finalize(st1, 2)
    sc_d5, bi_d5 = finalize(st1, 3)
    sc_mx, bi_mx = finalize(st1, 4)
    sc_av, bi_av = finalize(st1, 5)
    sc_s3, bi_s3 = finalize(st2, 0)
    sc_s5, bi_s5 = finalize(st2, 1)

    total_bias = (alphas[0] * bi_mx + alphas[1] * bi_av
                  + alphas[3] * bi_s3 + alphas[4] * bi_s5
                  + alphas[5] * bi_d3 + alphas[6] * bi_d5)
    sb = jnp.stack([t2(alphas[3] * sc_s3), t2(alphas[4] * sc_s5),
                    t2(alphas[0] * sc_mx), t2(alphas[1] * sc_av),
                    jnp.full((2 * C,), alphas[2], f32), t2(total_bias),
                    t2(alphas[5] * sc_d3), t2(alphas[6] * sc_d5)])

    out = _pass3(xf, (y_s3, y_s5, y_d3, y_d5, y_mx, y_av), (sb, ident),
                 B=B, N=N, C=C, H=H, W=W, pad_s=pad_s)
    return out.reshape(N, C, H, W)


# speedup vs baseline: 2.1530x; 1.0433x over previous
import functools

import jax
import jax.numpy as jnp
from jax import lax
from jax.experimental import pallas as pl
from jax.experimental.pallas import tpu as pltpu

_EPS = 1e-5
_NEG = -3.0e38

_PARAMS_1D = pltpu.CompilerParams(
    dimension_semantics=("parallel",),
    vmem_limit_bytes=64 * 1024 * 1024,
)


def _full_spec(shape):
    nd = len(shape)
    return pl.BlockSpec(shape, lambda *_: (0,) * nd)


def _geometry(H, W):
    HW = H * W
    iota = lax.broadcasted_iota(jnp.int32, (HW, 1), 0)
    if W & (W - 1) == 0:
        c = iota & (W - 1)
        r = iota >> (W.bit_length() - 1)
    else:
        c = iota % W
        r = iota // W
    cnt = ((1 + (r > 0).astype(jnp.int32) + (r < H - 1).astype(jnp.int32))
           * (1 + (c > 0).astype(jnp.int32) + (c < W - 1).astype(jnp.int32)))
    inv_cnt = 1.0 / cnt.astype(jnp.float32)
    return c, inv_cnt


def _col_mask(c, dx, W):
    if dx == 0:
        return None
    return (c >= -dx) if dx < 0 else (c < W - dx)


def _fill_shifted(dst, base, dx, border, mask, *, pad_s, HW):
    dst[:pad_s, :] = jnp.full((pad_s, dst.shape[1]), border, jnp.float32)
    dst[pad_s + HW:, :] = jnp.full((pad_s, dst.shape[1]), border, jnp.float32)
    v = base[pad_s + dx:pad_s + dx + HW, :]
    if mask is not None:
        v = jnp.where(mask, v, border)
    dst[pad_s:pad_s + HW, :] = v


def _fill_base(dst, interior, border, *, pad_s, HW):
    dst[:pad_s, :] = jnp.full((pad_s, dst.shape[1]), border, jnp.float32)
    dst[pad_s + HW:, :] = jnp.full((pad_s, dst.shape[1]), border, jnp.float32)
    dst[pad_s:pad_s + HW, :] = interior


def _conv_t(copies, wdw_ref, K, dil, *, pad_s, W, HW):
    half = (K // 2) * dil
    acc = None
    for kh in range(K):
        dy = kh * dil - half
        off = pad_s + dy * W
        for kw in range(K):
            dx = kw * dil - half
            v = copies[dx][off:off + HW, :]
            t = v * wdw_ref[kh * K + kw:kh * K + kw + 1, :]
            acc = t if acc is None else acc + t
    return acc


def _pool_t(copies, op, *, pad_s, W, HW):
    acc = None
    for dy in (-1, 0, 1):
        off = pad_s + dy * W
        for dx in (-1, 0, 1):
            v = copies[dx][off:off + HW, :]
            acc = v if acc is None else op(acc, v)
    return acc


def _transpose_in(x_ref, p, ident, C, HW):
    x2 = x_ref[2 * p:2 * p + 2].reshape(2 * C, HW)
    return lax.dot_general(x2, ident[...], (((0,), (0,)), ((), ())),
                           preferred_element_type=jnp.float32)


def _sums(y, ones_row):
    s = lax.dot_general(ones_row, y, (((1,), (0,)), ((), ())),
                        preferred_element_type=jnp.float32)
    ss = lax.dot_general(ones_row, y * y, (((1,), (0,)), ((), ())),
                         preferred_element_type=jnp.float32)
    return s, ss


def _p1_kernel(x_ref,
               wdw_s3, wpw_s3, wdw_s5, wpw_s5,
               wdw_d3, wpw_d3, wdw_d5, wpw_d5, ident,
               o_s3a, o_s5a, o_d3, o_d5, o_mx, o_av, stats_ref,
               *scr,
               B, C, H, W, pad_s):
    HW = H * W
    cidx, inv_cnt = _geometry(H, W)
    ones_row = jnp.ones((1, HW), jnp.float32)
    geo = dict(pad_s=pad_s, W=W, HW=HW)
    r_cp = {0: scr[0], 1: scr[1], -1: scr[2], 2: scr[3], -2: scr[4],
            4: scr[5], -4: scr[6]}
    a_cp = {0: scr[7], 1: scr[8], -1: scr[9]}
    m_cp = {0: scr[10], 1: scr[11], -1: scr[12]}

    for p in range(B // 2):
        xt = _transpose_in(x_ref, p, ident, C, HW)
        _fill_base(scr[0], jnp.maximum(xt, 0.0), 0.0, pad_s=pad_s, HW=HW)
        _fill_base(scr[7], xt, 0.0, pad_s=pad_s, HW=HW)
        _fill_base(scr[10], xt, _NEG, pad_s=pad_s, HW=HW)
        for dx in (1, -1, 2, -2, 4, -4):
            _fill_shifted(r_cp[dx], scr[0], dx, 0.0, _col_mask(cidx, dx, W),
                          pad_s=pad_s, HW=HW)
        for dx in (1, -1):
            _fill_shifted(a_cp[dx], scr[7], dx, 0.0, _col_mask(cidx, dx, W),
                          pad_s=pad_s, HW=HW)
            _fill_shifted(m_cp[dx], scr[10], dx, _NEG,
                          _col_mask(cidx, dx, W), pad_s=pad_s, HW=HW)

        for j, (o_ref, wdw, wpw, K, dil) in enumerate(
                ((o_s3a, wdw_s3, wpw_s3, 3, 1),
                 (o_s5a, wdw_s5, wpw_s5, 5, 1),
                 (o_d3, wdw_d3, wpw_d3, 3, 2),
                 (o_d5, wdw_d5, wpw_d5, 5, 2))):
            dw = _conv_t(r_cp, wdw, K, dil, **geo)
            y = jnp.dot(dw, wpw[...], preferred_element_type=jnp.float32)
            s, ss = _sums(y, ones_row)
            stats_ref[0, p, 2 * j:2 * j + 1, :] = s
            stats_ref[0, p, 2 * j + 1:2 * j + 2, :] = ss
            o_ref[0, p] = y.astype(jnp.bfloat16)

        mx = _pool_t(m_cp, jnp.maximum, **geo)
        av = _pool_t(a_cp, jnp.add, **geo) * inv_cnt
        for j, o_ref, y in ((4, o_mx, mx), (5, o_av, av)):
            s, ss = _sums(y, ones_row)
            stats_ref[0, p, 2 * j:2 * j + 1, :] = s
            stats_ref[0, p, 2 * j + 1:2 * j + 2, :] = ss
            o_ref[0, p] = y.astype(jnp.bfloat16)


def _pass1(xf, weights, *, B, N, C, H, W, pad_s):
    HW = H * W
    G = N // B
    P = B // 2
    PADT = HW + 2 * pad_s
    img_spec = pl.BlockSpec((B, C, HW), lambda n: (n, 0, 0))
    t_spec = pl.BlockSpec((1, P, HW, 2 * C), lambda n: (n, 0, 0, 0))
    return pl.pallas_call(
        functools.partial(_p1_kernel, B=B, C=C, H=H, W=W, pad_s=pad_s),
        grid=(G,),
        in_specs=[img_spec] + [_full_spec(w.shape) for w in weights],
        out_specs=[t_spec] * 6
        + [pl.BlockSpec((1, P, 16, 2 * C), lambda n: (n, 0, 0, 0))],
        out_shape=[jax.ShapeDtypeStruct((G, P, HW, 2 * C), jnp.bfloat16)] * 6
        + [jax.ShapeDtypeStruct((G, P, 16, 2 * C), jnp.float32)],
        scratch_shapes=[pltpu.VMEM((PADT, 2 * C), jnp.float32)] * 13,
        compiler_params=_PARAMS_1D,
    )(xf, *weights)


def _p2_kernel(y3_ref, y5_ref,
               wdw2_s3, wpw2_s3, wdw2_s5, wpw2_s5, bn_ref,
               o_s3, o_s5, stats_ref,
               *scr,
               B, C, H, W, pad_s):
    HW = H * W
    cidx, _ = _geometry(H, W)
    ones_row = jnp.ones((1, HW), jnp.float32)
    geo = dict(pad_s=pad_s, W=W, HW=HW)

    for p in range(B // 2):
        for j, (y_ref, wdw, wpw, K, o_ref) in enumerate(
                ((y3_ref, wdw2_s3, wpw2_s3, 3, o_s3),
                 (y5_ref, wdw2_s5, wpw2_s5, 5, o_s5))):
            y = y_ref[0, p].astype(jnp.float32)
            a = jnp.maximum(y * bn_ref[2 * j:2 * j + 1, :]
                            + bn_ref[2 * j + 1:2 * j + 2, :], 0.0)
            cp = {0: scr[0], 1: scr[1], -1: scr[2], 2: scr[3], -2: scr[4]}
            _fill_base(scr[0], a, 0.0, pad_s=pad_s, HW=HW)
            dxs = (1, -1) if K == 3 else (1, -1, 2, -2)
            for dx in dxs:
                _fill_shifted(cp[dx], scr[0], dx, 0.0,
                              _col_mask(cidx, dx, W), pad_s=pad_s, HW=HW)
            dw = _conv_t(cp, wdw, K, 1, **geo)
            out = jnp.dot(dw, wpw[...], preferred_element_type=jnp.float32)
            s, ss = _sums(out, ones_row)
            stats_ref[0, p, 2 * j:2 * j + 1, :] = s
            stats_ref[0, p, 2 * j + 1:2 * j + 2, :] = ss
            o_ref[0, p] = out.astype(jnp.bfloat16)


def _pass2(y3, y5, weights, *, B, N, C, H, W, pad_s):
    HW = H * W
    G = N // B
    P = B // 2
    PADT = HW + 2 * pad_s
    t_spec = pl.BlockSpec((1, P, HW, 2 * C), lambda n: (n, 0, 0, 0))
    return pl.pallas_call(
        functools.partial(_p2_kernel, B=B, C=C, H=H, W=W, pad_s=pad_s),
        grid=(G,),
        in_specs=[t_spec, t_spec] + [_full_spec(w.shape) for w in weights],
        out_specs=[t_spec, t_spec,
                   pl.BlockSpec((1, P, 8, 2 * C), lambda n: (n, 0, 0, 0))],
        out_shape=[jax.ShapeDtypeStruct((G, P, HW, 2 * C), jnp.bfloat16)] * 2
        + [jax.ShapeDtypeStruct((G, P, 8, 2 * C), jnp.float32)],
        scratch_shapes=[pltpu.VMEM((PADT, 2 * C), jnp.float32)] * 5,
        compiler_params=_PARAMS_1D,
    )(y3, y5, *weights)


def _p3_kernel(x_ref, s3_ref, s5_ref, d3_ref, d5_ref, mx_ref, av_ref,
               sb_ref, ident, o_ref,
               *, B, C, H, W, pad_s):
    HW = H * W
    sb = sb_ref[...]
    for p in range(B // 2):
        xt = _transpose_in(x_ref, p, ident, C, HW)
        acc = xt * sb[4:5, :] + sb[5:6, :]
        for row, ref in ((0, s3_ref), (1, s5_ref), (6, d3_ref), (7, d5_ref),
                         (2, mx_ref), (3, av_ref)):
            acc = acc + ref[0, p].astype(jnp.float32) * sb[row:row + 1, :]
        out2 = lax.dot_general(ident[...], acc, (((1,), (1,)), ((), ())),
                               preferred_element_type=jnp.float32)
        o_ref[2 * p] = out2[:C]
        o_ref[2 * p + 1] = out2[C:]


def _pass3(xf, branches, weights, *, B, N, C, H, W, pad_s):
    HW = H * W
    G = N // B
    P = B // 2
    img_spec = pl.BlockSpec((B, C, HW), lambda n: (n, 0, 0))
    t_spec = pl.BlockSpec((1, P, HW, 2 * C), lambda n: (n, 0, 0, 0))
    return pl.pallas_call(
        functools.partial(_p3_kernel, B=B, C=C, H=H, W=W, pad_s=pad_s),
        grid=(G,),
        in_specs=[img_spec] + [t_spec] * 6
        + [_full_spec(w.shape) for w in weights],
        out_specs=img_spec,
        out_shape=jax.ShapeDtypeStruct((N, C, HW), jnp.float32),
        compiler_params=_PARAMS_1D,
    )(xf, *branches, *weights)


def kernel(x, sep3_dw1, sep3_pw1, sep3_dw2, sep3_pw2,
           sep5_dw1, sep5_pw1, sep5_dw2, sep5_pw2,
           dil3_dw, dil3_pw, dil5_dw, dil5_pw, alphas):
    N, C, H, W = x.shape
    HW = H * W
    f32 = jnp.float32
    x = x.astype(f32)
    xf = x.reshape(N, C, HW)
    alphas = jnp.asarray(alphas, f32)

    B = 8 if N % 8 == 0 else (4 if N % 4 == 0 else 2)
    pad_s = ((4 * W + 4 + 7) // 8) * 8

    ident = jnp.eye(2 * C, dtype=f32)

    def dwT(a):
        return jnp.tile(a.reshape(C, -1).astype(f32).T, (1, 2))

    def pwT(a, scale=None):
        m = a[:, :, 0, 0].astype(f32).T
        if scale is not None:
            m = m * scale[None, :]
        z = jnp.zeros((C, C), f32)
        return jnp.concatenate(
            [jnp.concatenate([m, z], axis=1),
             jnp.concatenate([z, m], axis=1)], axis=0)

    w1 = (dwT(sep3_dw1), pwT(sep3_pw1), dwT(sep5_dw1), pwT(sep5_pw1),
          dwT(dil3_dw), pwT(dil3_pw), dwT(dil5_dw), pwT(dil5_pw), ident)
    (y_s3a, y_s5a, y_d3, y_d5, y_mx, y_av, stats1) = _pass1(
        xf, w1, B=B, N=N, C=C, H=H, W=W, pad_s=pad_s)

    total = jnp.float32(N * HW)
    st1 = jnp.sum(stats1, axis=(0, 1)).reshape(16, 2, C).sum(axis=1)

    def finalize(st, j):
        s, ss = st[2 * j], st[2 * j + 1]
        m = s / total
        v = jnp.maximum(ss / total - m * m, 0.0)
        sc = lax.rsqrt(v + _EPS)
        return sc, -m * sc

    t2 = lambda v: jnp.tile(v, 2)
    sc_s3a, bi_s3a = finalize(st1, 0)
    sc_s5a, bi_s5a = finalize(st1, 1)
    bn_mid = jnp.stack([t2(sc_s3a), t2(bi_s3a), t2(sc_s5a), t2(bi_s5a)])

    w2 = (dwT(sep3_dw2), pwT(sep3_pw2), dwT(sep5_dw2), pwT(sep5_pw2), bn_mid)
    y_s3, y_s5, stats2 = _pass2(y_s3a, y_s5a, w2, B=B, N=N, C=C, H=H, W=W,
                                pad_s=pad_s)
    st2 = jnp.sum(stats2, axis=(0, 1)).reshape(8, 2, C).sum(axis=1)

    sc_d3, bi_d3 = finalize(st1, 2)
    sc_d5, bi_d5 = finalize(st1, 3)
    sc_mx, bi_mx = finalize(st1, 4)
    sc_av, bi_av = finalize(st1, 5)
    sc_s3, bi_s3 = finalize(st2, 0)
    sc_s5, bi_s5 = finalize(st2, 1)

    total_bias = (alphas[0] * bi_mx + alphas[1] * bi_av
                  + alphas[3] * bi_s3 + alphas[4] * bi_s5
                  + alphas[5] * bi_d3 + alphas[6] * bi_d5)
    sb = jnp.stack([t2(alphas[3] * sc_s3), t2(alphas[4] * sc_s5),
                    t2(alphas[0] * sc_mx), t2(alphas[1] * sc_av),
                    jnp.full((2 * C,), alphas[2], f32), t2(total_bias),
                    t2(alphas[5] * sc_d3), t2(alphas[6] * sc_d5)])

    out = _pass3(xf, (y_s3, y_s5, y_d3, y_d5, y_mx, y_av), (sb, ident),
                 B=B, N=N, C=C, H=H, W=W, pad_s=pad_s)
    return out.reshape(N, C, H, W)
